# baseline (device time: 55818 ns/iter reference)
import jax
import jax.numpy as jnp
from jax import lax
from jax.experimental import pallas as pl
from jax.experimental.pallas import tpu as pltpu

B, S, N = 4, 512, 1024
K = 512
S_HALF = S // 2


def kernel(O, Wo):
    O3 = O.reshape(B, S, K)

    def body(o_ref, wo_ref, out_ref, comm_ref, send_sems, recv_sems):
        my_x = lax.axis_index("x")
        my_y = lax.axis_index("y")
        my_z = lax.axis_index("z")
        other_y = 1 - my_y
        partner = (my_x, other_y, my_z)

        barrier = pltpu.get_barrier_semaphore()
        pl.semaphore_signal(
            barrier, inc=1, device_id=partner,
            device_id_type=pl.DeviceIdType.MESH,
        )
        pl.semaphore_wait(barrier, 1)

        my_start = my_y * S_HALF
        other_start = other_y * S_HALF

        rdmas = []
        for b in range(B):
            part = jnp.dot(
                o_ref[b, pl.ds(other_start, S_HALF), :], wo_ref[:, :],
                preferred_element_type=jnp.float32,
            )
            comm_ref[0, b] = part
            rdma = pltpu.make_async_remote_copy(
                src_ref=comm_ref.at[0, b],
                dst_ref=comm_ref.at[1, b],
                send_sem=send_sems.at[b],
                recv_sem=recv_sems.at[b],
                device_id=partner,
                device_id_type=pl.DeviceIdType.MESH,
            )
            rdma.start()
            rdmas.append(rdma)

        for b in range(B):
            own = jnp.dot(
                o_ref[b, pl.ds(my_start, S_HALF), :], wo_ref[:, :],
                preferred_element_type=jnp.float32,
            )
            rdmas[b].wait_recv()
            out_ref[b] = own + comm_ref[1, b]

        for b in range(B):
            rdmas[b].wait_send()

    return pl.pallas_call(
        body,
        out_shape=jax.ShapeDtypeStruct((B, S_HALF, N), jnp.float32),
        in_specs=[
            pl.BlockSpec(memory_space=pltpu.VMEM),
            pl.BlockSpec(memory_space=pltpu.VMEM),
        ],
        out_specs=pl.BlockSpec(memory_space=pltpu.VMEM),
        scratch_shapes=[
            pltpu.VMEM((2, B, S_HALF, N), jnp.float32),
            pltpu.SemaphoreType.DMA((B,)),
            pltpu.SemaphoreType.DMA((B,)),
        ],
        compiler_params=pltpu.CompilerParams(collective_id=0),
    )(O3, Wo)


# device time: 33061 ns/iter; 1.6883x vs baseline; 1.6883x over previous
import jax
import jax.numpy as jnp
from jax import lax
from jax.experimental import pallas as pl
from jax.experimental.pallas import tpu as pltpu

B, S, N = 4, 512, 1024
K = 512
S_HALF = S // 2
ROWS = 128
CPB = S_HALF // ROWS
NCHUNK = B * CPB


def kernel(O, Wo):
    O3 = O.reshape(B, S, K)

    def body(o_ref, wo_ref, out_ref, comm_ref, send_sems, recv_sems):
        my_x = lax.axis_index("x")
        my_y = lax.axis_index("y")
        my_z = lax.axis_index("z")
        other_y = 1 - my_y
        partner = (my_x, other_y, my_z)

        barrier = pltpu.get_barrier_semaphore()
        pl.semaphore_signal(
            barrier, inc=1, device_id=partner,
            device_id_type=pl.DeviceIdType.MESH,
        )
        pl.semaphore_wait(barrier, 1)

        my_start = my_y * S_HALF
        other_start = other_y * S_HALF

        rdmas = []
        for b in range(B):
            for c in range(CPB):
                i = b * CPB + c
                part = jnp.dot(
                    o_ref[b, pl.ds(other_start + c * ROWS, ROWS), :],
                    wo_ref[:, :],
                    preferred_element_type=jnp.float32,
                )
                comm_ref[0, i] = part.astype(jnp.bfloat16)
                rdma = pltpu.make_async_remote_copy(
                    src_ref=comm_ref.at[0, i],
                    dst_ref=comm_ref.at[1, i],
                    send_sem=send_sems.at[i],
                    recv_sem=recv_sems.at[i],
                    device_id=partner,
                    device_id_type=pl.DeviceIdType.MESH,
                )
                rdma.start()
                rdmas.append(rdma)

        for b in range(B):
            for c in range(CPB):
                i = b * CPB + c
                own = jnp.dot(
                    o_ref[b, pl.ds(my_start + c * ROWS, ROWS), :],
                    wo_ref[:, :],
                    preferred_element_type=jnp.float32,
                )
                rdmas[i].wait_recv()
                out_ref[b, pl.ds(c * ROWS, ROWS), :] = (
                    own + comm_ref[1, i].astype(jnp.float32)
                )

        for i in range(NCHUNK):
            rdmas[i].wait_send()

    return pl.pallas_call(
        body,
        out_shape=jax.ShapeDtypeStruct((B, S_HALF, N), jnp.float32),
        in_specs=[
            pl.BlockSpec(memory_space=pltpu.VMEM),
            pl.BlockSpec(memory_space=pltpu.VMEM),
        ],
        out_specs=pl.BlockSpec(memory_space=pltpu.VMEM),
        scratch_shapes=[
            pltpu.VMEM((2, NCHUNK, ROWS, N), jnp.bfloat16),
            pltpu.SemaphoreType.DMA((NCHUNK,)),
            pltpu.SemaphoreType.DMA((NCHUNK,)),
        ],
        compiler_params=pltpu.CompilerParams(collective_id=0),
    )(O3, Wo)


# device time: 27228 ns/iter; 2.0500x vs baseline; 1.2142x over previous
import jax
import jax.numpy as jnp
from jax import lax
from jax.experimental import pallas as pl
from jax.experimental.pallas import tpu as pltpu

B, S, N = 4, 512, 1024
K = 512
S_HALF = S // 2
ROWS = 128


def kernel(O, Wo):
    O3 = O.reshape(B, S, K)

    def body(o_ref, wo_ref, out_ref, ysend, yrecv, zrecv,
             ysend_sems, yrecv_sems, zsend_sems, zrecv_sems):
        my_x = lax.axis_index("x")
        my_y = lax.axis_index("y")
        my_z = lax.axis_index("z")
        other_y = 1 - my_y
        ypartner = (my_x, other_y, my_z)
        zneighbor = (my_x, my_y, 1 - my_z)

        barrier = pltpu.get_barrier_semaphore()
        for nbr in (ypartner, zneighbor):
            pl.semaphore_signal(
                barrier, inc=1, device_id=nbr,
                device_id_type=pl.DeviceIdType.MESH,
            )
        pl.semaphore_wait(barrier, 2)

        my_start = my_y * S_HALF
        other_start = other_y * S_HALF
        zc = my_z * ROWS
        oc = (1 - my_z) * ROWS

        y_rdmas = []
        for b in range(B):
            part = jnp.dot(
                o_ref[b, pl.ds(other_start + zc, ROWS), :], wo_ref[:, :],
                preferred_element_type=jnp.float32,
            )
            ysend[b] = part.astype(jnp.bfloat16)
            r = pltpu.make_async_remote_copy(
                src_ref=ysend.at[b],
                dst_ref=yrecv.at[b],
                send_sem=ysend_sems.at[b],
                recv_sem=yrecv_sems.at[b],
                device_id=ypartner,
                device_id_type=pl.DeviceIdType.MESH,
            )
            r.start()
            y_rdmas.append(r)

        z_rdmas = []
        for b in range(B):
            own = jnp.dot(
                o_ref[b, pl.ds(my_start + zc, ROWS), :], wo_ref[:, :],
                preferred_element_type=jnp.float32,
            )
            y_rdmas[b].wait_recv()
            zr = pltpu.make_async_remote_copy(
                src_ref=yrecv.at[b],
                dst_ref=zrecv.at[b],
                send_sem=zsend_sems.at[b],
                recv_sem=zrecv_sems.at[b],
                device_id=zneighbor,
                device_id_type=pl.DeviceIdType.MESH,
            )
            zr.start()
            z_rdmas.append(zr)
            out_ref[b, pl.ds(zc, ROWS), :] = own + yrecv[b].astype(jnp.float32)

        for b in range(B):
            own = jnp.dot(
                o_ref[b, pl.ds(my_start + oc, ROWS), :], wo_ref[:, :],
                preferred_element_type=jnp.float32,
            )
            z_rdmas[b].wait_recv()
            out_ref[b, pl.ds(oc, ROWS), :] = own + zrecv[b].astype(jnp.float32)

        for b in range(B):
            y_rdmas[b].wait_send()
            z_rdmas[b].wait_send()

    return pl.pallas_call(
        body,
        out_shape=jax.ShapeDtypeStruct((B, S_HALF, N), jnp.float32),
        in_specs=[
            pl.BlockSpec(memory_space=pltpu.VMEM),
            pl.BlockSpec(memory_space=pltpu.VMEM),
        ],
        out_specs=pl.BlockSpec(memory_space=pltpu.VMEM),
        scratch_shapes=[
            pltpu.VMEM((B, ROWS, N), jnp.bfloat16),
            pltpu.VMEM((B, ROWS, N), jnp.bfloat16),
            pltpu.VMEM((B, ROWS, N), jnp.bfloat16),
            pltpu.SemaphoreType.DMA((B,)),
            pltpu.SemaphoreType.DMA((B,)),
            pltpu.SemaphoreType.DMA((B,)),
            pltpu.SemaphoreType.DMA((B,)),
        ],
        compiler_params=pltpu.CompilerParams(collective_id=0),
    )(O3, Wo)
